# Initial kernel scaffold; baseline (speedup 1.0000x reference)
import jax
import jax.numpy as jnp
from jax.experimental import pallas as pl


def kernel(ids, table):
    raise NotImplementedError("write your pallas kernel here")


# speedup vs baseline: 4.0727x; 4.0727x over previous
import functools

import jax
import jax.numpy as jnp
from jax import lax
from jax.experimental import pallas as pl
from jax.experimental.pallas import tpu as pltpu
from jax.experimental.pallas import tpu_sc as plsc

_LANES = 16
_NC, _NS = 2, 16
_NW = _NC * _NS


def _make_encoder(B, V, D, LP, rows_per_w):
    half = LP // 2
    mesh = plsc.VectorSubcoreMesh(core_axis_name="c", subcore_axis_name="s")

    @functools.partial(
        pl.kernel,
        mesh=mesh,
        out_type=jax.ShapeDtypeStruct((B, D), jnp.float32),
        compiler_params=pltpu.CompilerParams(
            use_tc_tiling_on_sc=False, needs_layout_passes=False
        ),
        scratch_types=[
            pltpu.VMEM((rows_per_w, LP), jnp.int32),
            pltpu.VMEM((LP, D), jnp.float32),
            pltpu.VMEM((rows_per_w, D), jnp.float32),
            pltpu.VMEM((D,), jnp.float32),
        ],
    )
    def enc(ids_hbm, table_hbm, out_hbm, ids_v, rows_v, out_v, t0_v):
        wid = lax.axis_index("s") * _NC + lax.axis_index("c")
        base = wid * rows_per_w
        pltpu.sync_copy(table_hbm.at[0], t0_v)
        pltpu.sync_copy(ids_hbm.at[pl.ds(base, rows_per_w)], ids_v)

        @pl.loop(0, rows_per_w)
        def _row(r):
            pltpu.sync_copy(
                table_hbm.at[ids_v.at[r, pl.ds(0, half)]],
                rows_v.at[pl.ds(0, half)],
            )
            pltpu.sync_copy(
                table_hbm.at[ids_v.at[r, pl.ds(half, half)]],
                rows_v.at[pl.ds(half, half)],
            )

            nz = jnp.zeros((_LANES,), jnp.int32)
            for j in range(LP // _LANES):
                v = ids_v[r, pl.ds(j * _LANES, _LANES)]
                nz = nz + plsc.all_reduce_population_count(v == 0)
            nzf = nz.astype(jnp.float32)

            def body(t, accs):
                return tuple(
                    accs[k] + rows_v[t, pl.ds(k * _LANES, _LANES)]
                    for k in range(D // _LANES)
                )

            zero = jnp.zeros((_LANES,), jnp.float32)
            accs = lax.fori_loop(0, LP, body, (zero,) * (D // _LANES))

            denom = jnp.maximum(jnp.float32(LP) - nzf, 1.0)
            scale = 1.0 / denom
            for k in range(D // _LANES):
                t0k = t0_v[pl.ds(k * _LANES, _LANES)]
                out_v[r, pl.ds(k * _LANES, _LANES)] = (accs[k] - nzf * t0k) * scale

        pltpu.sync_copy(out_v, out_hbm.at[pl.ds(base, rows_per_w)])

    return enc


def kernel(ids, table):
    B, S = ids.shape
    V, D = table.shape
    LP = ((S + _LANES - 1) // _LANES) * _LANES
    if (LP // 2) % 8 != 0:
        LP += _LANES
    ids_p = ids.astype(jnp.int32)
    if LP != S:
        ids_p = jnp.pad(ids_p, ((0, 0), (0, LP - S)))
    rows_per_w = B // _NW
    enc = _make_encoder(B, V, D, LP, rows_per_w)
    return enc(ids_p, table)
